# baseline (speedup 1.0000x reference)
import jax
import jax.numpy as jnp
from jax.experimental import pallas as pl
from jax.experimental.pallas import tpu as pltpu
from jax.experimental.pallas import tpu_sc as plsc

_EMBED_DIM = 64
_N_EMBED = 1024
_BLOCK_M = 512
_CH = 128

_NC = 2
_NS = 16
_NW = _NC * _NS
_IDXC = 128


def _argmin_block(x_ref, emb_ref, idx_ref, e2_ref):
    x = x_ref[...]
    emb = emb_ref[...]

    @pl.when(pl.program_id(0) == 0)
    def _():
        e2_ref[...] = jnp.sum(emb * emb, axis=0, keepdims=True)

    x2 = jnp.sum(x * x, axis=1, keepdims=True)
    e2 = e2_ref[...]
    dot = jnp.dot(x, emb, preferred_element_type=jnp.float32)

    iota = jax.lax.broadcasted_iota(jnp.int32, (_BLOCK_M, _CH), 1)
    cm = (x2 - 2.0 * dot[:, 0:_CH]) + e2[:, 0:_CH]
    js = iota
    for k in range(1, _N_EMBED // _CH):
        dk = (x2 - 2.0 * dot[:, k * _CH:(k + 1) * _CH]) + e2[:, k * _CH:(k + 1) * _CH]
        lt = dk < cm
        cm = jnp.where(lt, dk, cm)
        js = jnp.where(lt, iota + (k * _CH), js)
    cmt = cm.T
    jst = js.T
    mt = jnp.min(cmt, axis=0, keepdims=True)
    idx = jnp.min(jnp.where(cmt == mt, jst, _N_EMBED), axis=0)
    idx_ref[0, 0, :] = idx


def _gather_body(table_hbm, idx_hbm, quant_hbm, idx_v, rows_v, sem):
    wid = jax.lax.axis_index("s") * _NC + jax.lax.axis_index("c")
    pltpu.sync_copy(idx_hbm.at[wid], idx_v)
    kc = idx_v.shape[0]
    for j in range(kc):
        pltpu.async_copy(table_hbm.at[idx_v.at[j]], rows_v.at[j], sem).wait()
    base = wid * kc * _IDXC
    for j in range(kc):
        pltpu.sync_copy(rows_v.at[j],
                        quant_hbm.at[pl.ds(base + j * _IDXC, _IDXC)])


def kernel(inputs, embedding):
    lead_shape = inputs.shape[:-1]
    flat = inputs.reshape(-1, _EMBED_DIM)
    n_rows = flat.shape[0]
    grid = n_rows // _BLOCK_M
    embt = embedding.T

    idx3 = pl.pallas_call(
        _argmin_block,
        grid=(grid,),
        in_specs=[
            pl.BlockSpec((_BLOCK_M, _EMBED_DIM), lambda i: (i, 0)),
            pl.BlockSpec((_EMBED_DIM, _N_EMBED), lambda i: (0, 0)),
        ],
        out_specs=pl.BlockSpec((1, 1, _BLOCK_M), lambda i: (i, 0, 0)),
        out_shape=jax.ShapeDtypeStruct((grid, 1, _BLOCK_M), jnp.int32),
        scratch_shapes=[pltpu.VMEM((1, _N_EMBED), jnp.float32)],
    )(flat, embedding)

    kc = n_rows // (_NW * _IDXC)
    idx_sc = idx3.reshape(_NW, kc, _IDXC)
    table = jnp.pad(embt, ((0, 0), (0, 2 * _EMBED_DIM - embt.shape[1])))

    sc_gather = pl.kernel(
        _gather_body,
        mesh=plsc.VectorSubcoreMesh(core_axis_name="c", subcore_axis_name="s"),
        out_type=jax.ShapeDtypeStruct((n_rows, 2 * _EMBED_DIM), jnp.float32),
        scratch_types=[
            pltpu.VMEM((kc, _IDXC), jnp.int32),
            pltpu.VMEM((kc, _IDXC, 2 * _EMBED_DIM), jnp.float32),
            pltpu.SemaphoreType.DMA,
        ],
    )
    quant = sc_gather(table, idx_sc)

    quantized = quant[:, :_EMBED_DIM].reshape(inputs.shape)
    codes_out = jnp.concatenate([inputs, quantized], axis=-1)
    encoding_indices = idx3.reshape(lead_shape)
    return (quantized, codes_out, encoding_indices)


# speedup vs baseline: 1.4337x; 1.4337x over previous
import jax
import jax.numpy as jnp
from jax.experimental import pallas as pl
from jax.experimental.pallas import tpu as pltpu

_EMBED_DIM = 64
_N_EMBED = 1024
_BLOCK_M = 512
_CH = 128


def _vq_block(x_ref, emb_ref, emb2_ref, embt_ref,
              idx_ref, quant_ref, codes_ref, e2_ref):
    x = x_ref[...]
    emb2 = emb2_ref[...]

    @pl.when(pl.program_id(0) == 0)
    def _():
        emb = emb_ref[...]
        e2_ref[...] = jnp.sum(emb * emb, axis=0, keepdims=True)

    x2 = jnp.sum(x * x, axis=1, keepdims=True)
    e2 = e2_ref[...]
    dot2 = jnp.dot(x, emb2, preferred_element_type=jnp.float32)

    iota = jax.lax.broadcasted_iota(jnp.int32, (_BLOCK_M, _CH), 1)
    cm = (x2 - dot2[:, 0:_CH]) + e2[:, 0:_CH]
    js = iota
    for k in range(1, _N_EMBED // _CH):
        dk = (x2 - dot2[:, k * _CH:(k + 1) * _CH]) + e2[:, k * _CH:(k + 1) * _CH]
        lt = dk < cm
        cm = jnp.where(lt, dk, cm)
        js = jnp.where(lt, iota + (k * _CH), js)
    cmt = cm.T
    jst = js.T
    mt = jnp.min(cmt, axis=0, keepdims=True)
    idx = jnp.min(jnp.where(cmt == mt, jst, _N_EMBED), axis=0)
    idx_ref[0, 0, :] = idx

    full_iota = jax.lax.broadcasted_iota(jnp.int32, (_BLOCK_M, _N_EMBED), 1)
    onehot = (full_iota == idx[:, None]).astype(jnp.float32)
    embt = embt_ref[...]
    q = jnp.dot(onehot, embt, preferred_element_type=jnp.float32)
    quant_ref[...] = x + (q - x)
    codes_ref[...] = jnp.concatenate([x, q], axis=1)


def kernel(inputs, embedding):
    lead_shape = inputs.shape[:-1]
    flat = inputs.reshape(-1, _EMBED_DIM)
    n_rows = flat.shape[0]
    grid = n_rows // _BLOCK_M
    embt = embedding.T
    emb2 = embedding + embedding

    idx3, quant, codes = pl.pallas_call(
        _vq_block,
        grid=(grid,),
        in_specs=[
            pl.BlockSpec((_BLOCK_M, _EMBED_DIM), lambda i: (i, 0)),
            pl.BlockSpec((_EMBED_DIM, _N_EMBED), lambda i: (0, 0)),
            pl.BlockSpec((_EMBED_DIM, _N_EMBED), lambda i: (0, 0)),
            pl.BlockSpec((_N_EMBED, _EMBED_DIM), lambda i: (0, 0)),
        ],
        out_specs=[
            pl.BlockSpec((1, 1, _BLOCK_M), lambda i: (i, 0, 0)),
            pl.BlockSpec((_BLOCK_M, _EMBED_DIM), lambda i: (i, 0)),
            pl.BlockSpec((_BLOCK_M, 2 * _EMBED_DIM), lambda i: (i, 0)),
        ],
        out_shape=[
            jax.ShapeDtypeStruct((grid, 1, _BLOCK_M), jnp.int32),
            jax.ShapeDtypeStruct((n_rows, _EMBED_DIM), jnp.float32),
            jax.ShapeDtypeStruct((n_rows, 2 * _EMBED_DIM), jnp.float32),
        ],
        scratch_shapes=[pltpu.VMEM((1, _N_EMBED), jnp.float32)],
    )(flat, embedding, emb2, embt)

    quantized = quant.reshape(inputs.shape)
    codes_out = codes.reshape(lead_shape + (2 * _EMBED_DIM,))
    encoding_indices = idx3.reshape(lead_shape)
    return (quantized, codes_out, encoding_indices)


# speedup vs baseline: 1.8366x; 1.2810x over previous
import jax
import jax.numpy as jnp
from jax.experimental import pallas as pl
from jax.experimental.pallas import tpu as pltpu

_EMBED_DIM = 64
_N_EMBED = 1024
_BLOCK_M = 1024
_RCH = 128


def _vq_block(x_ref, x2_ref, emb_ref, emb2t_ref, embt_ref,
              idx_ref, quant_ref, codes_ref, e2b_ref):
    x = x_ref[...]
    emb2t = emb2t_ref[...]

    @pl.when(pl.program_id(0) == 0)
    def _():
        emb = emb_ref[...]
        e2row = jnp.sum(emb * emb, axis=0, keepdims=True)
        e2b_ref[...] = jnp.broadcast_to(e2row.T, (_N_EMBED, _BLOCK_M))

    x2 = x2_ref[0, 0, :][None, :]
    xt = x.T
    dot2t = jnp.dot(emb2t, xt, preferred_element_type=jnp.float32)

    nch = _N_EMBED // _RCH
    e2b = e2b_ref[...]
    pairs = [
        ((x2 - dot2t[k * _RCH:(k + 1) * _RCH, :]) + e2b[k * _RCH:(k + 1) * _RCH, :],
         jnp.full((_RCH, _BLOCK_M), k, jnp.int32))
        for k in range(nch)
    ]
    while len(pairs) > 1:
        nxt = []
        for a, b in zip(pairs[0::2], pairs[1::2]):
            lt = b[0] < a[0]
            nxt.append((jnp.where(lt, b[0], a[0]), jnp.where(lt, b[1], a[1])))
        pairs = nxt
    cm, jc = pairs[0]
    jiota = jax.lax.broadcasted_iota(jnp.int32, (_RCH, _BLOCK_M), 0)
    js = jc * _RCH + jiota
    mt = jnp.min(cm, axis=0, keepdims=True)
    idx = jnp.min(jnp.where(cm == mt, js, _N_EMBED), axis=0)
    idx_ref[0, 0, :] = idx

    fiota = jax.lax.broadcasted_iota(jnp.int32, (_N_EMBED, _BLOCK_M), 0)
    onehot_t = (fiota == idx[None, :]).astype(jnp.float32)
    emb = emb_ref[...]
    qt = jnp.dot(emb, onehot_t, preferred_element_type=jnp.float32)
    q = qt.T
    quant_ref[...] = x + (q - x)
    codes_ref[...] = jnp.concatenate([x, q], axis=1)


def kernel(inputs, embedding):
    lead_shape = inputs.shape[:-1]
    flat = inputs.reshape(-1, _EMBED_DIM)
    n_rows = flat.shape[0]
    grid = n_rows // _BLOCK_M
    embt = embedding.T
    emb2t = embt + embt
    x2 = jnp.sum(flat * flat, axis=1).reshape(grid, 1, _BLOCK_M)

    idx3, quant, codes = pl.pallas_call(
        _vq_block,
        grid=(grid,),
        in_specs=[
            pl.BlockSpec((_BLOCK_M, _EMBED_DIM), lambda i: (i, 0)),
            pl.BlockSpec((1, 1, _BLOCK_M), lambda i: (i, 0, 0)),
            pl.BlockSpec((_EMBED_DIM, _N_EMBED), lambda i: (0, 0)),
            pl.BlockSpec((_N_EMBED, _EMBED_DIM), lambda i: (0, 0)),
            pl.BlockSpec((_N_EMBED, _EMBED_DIM), lambda i: (0, 0)),
        ],
        out_specs=[
            pl.BlockSpec((1, 1, _BLOCK_M), lambda i: (i, 0, 0)),
            pl.BlockSpec((_BLOCK_M, _EMBED_DIM), lambda i: (i, 0)),
            pl.BlockSpec((_BLOCK_M, 2 * _EMBED_DIM), lambda i: (i, 0)),
        ],
        out_shape=[
            jax.ShapeDtypeStruct((grid, 1, _BLOCK_M), jnp.int32),
            jax.ShapeDtypeStruct((n_rows, _EMBED_DIM), jnp.float32),
            jax.ShapeDtypeStruct((n_rows, 2 * _EMBED_DIM), jnp.float32),
        ],
        scratch_shapes=[pltpu.VMEM((_N_EMBED, _BLOCK_M), jnp.float32)],
    )(flat, x2, embedding, emb2t, embt)

    quantized = quant.reshape(inputs.shape)
    codes_out = codes.reshape(lead_shape + (2 * _EMBED_DIM,))
    encoding_indices = idx3.reshape(lead_shape)
    return (quantized, codes_out, encoding_indices)


# speedup vs baseline: 1.8868x; 1.0273x over previous
import jax
import jax.numpy as jnp
from jax.experimental import pallas as pl
from jax.experimental.pallas import tpu as pltpu

_EMBED_DIM = 64
_N_EMBED = 1024
_BLOCK_M = 1024
_RCH = 128


def _vq_block(x_ref, x2_ref, emb_ref, emb2t_ref,
              idx_ref, quant_ref, codes_ref,
              e2b_ref, idx_s_ref, x_s_ref):
    @pl.when(pl.program_id(0) == 0)
    def _():
        emb0 = emb_ref[...]
        e2row = jnp.sum(emb0 * emb0, axis=0, keepdims=True)
        e2b_ref[...] = jnp.broadcast_to(e2row.T, (_N_EMBED, _BLOCK_M))

    idxp = idx_s_ref[0, :]
    fiota = jax.lax.broadcasted_iota(jnp.int32, (_N_EMBED, _BLOCK_M), 0)
    onehot_t = (fiota == idxp[None, :]).astype(jnp.float32)
    emb = emb_ref[...]
    qt = jnp.dot(emb, onehot_t, preferred_element_type=jnp.float32)
    q = qt.T
    xp = x_s_ref[...]
    idx_ref[0, 0, :] = idxp
    quant_ref[...] = xp + (q - xp)
    codes_ref[...] = jnp.concatenate([xp, q], axis=1)

    x = x_ref[...]
    emb2t = emb2t_ref[...]
    x2 = x2_ref[0, 0, :][None, :]
    xt = x.T
    dot2t = jnp.dot(emb2t, xt, preferred_element_type=jnp.float32)

    nch = _N_EMBED // _RCH
    e2b = e2b_ref[...]
    pairs = [
        ((x2 - dot2t[k * _RCH:(k + 1) * _RCH, :]) + e2b[k * _RCH:(k + 1) * _RCH, :],
         jnp.full((_RCH, _BLOCK_M), k, jnp.int32))
        for k in range(nch)
    ]
    while len(pairs) > 1:
        nxt = []
        for a, b in zip(pairs[0::2], pairs[1::2]):
            lt = b[0] < a[0]
            nxt.append((jnp.where(lt, b[0], a[0]), jnp.where(lt, b[1], a[1])))
        pairs = nxt
    cm, jc = pairs[0]
    jiota = jax.lax.broadcasted_iota(jnp.int32, (_RCH, _BLOCK_M), 0)
    js = jc * _RCH + jiota
    mt = jnp.min(cm, axis=0, keepdims=True)
    idx = jnp.min(jnp.where(cm == mt, js, _N_EMBED), axis=0)
    idx_s_ref[0, :] = idx
    x_s_ref[...] = x


def kernel(inputs, embedding):
    lead_shape = inputs.shape[:-1]
    flat = inputs.reshape(-1, _EMBED_DIM)
    n_rows = flat.shape[0]
    grid = n_rows // _BLOCK_M
    embt = embedding.T
    emb2t = embt + embt
    x2 = jnp.sum(flat * flat, axis=1).reshape(grid, 1, _BLOCK_M)

    last = grid - 1
    idx3, quant, codes = pl.pallas_call(
        _vq_block,
        grid=(grid + 1,),
        in_specs=[
            pl.BlockSpec((_BLOCK_M, _EMBED_DIM),
                         lambda i: (jnp.minimum(i, last), 0)),
            pl.BlockSpec((1, 1, _BLOCK_M),
                         lambda i: (jnp.minimum(i, last), 0, 0)),
            pl.BlockSpec((_EMBED_DIM, _N_EMBED), lambda i: (0, 0)),
            pl.BlockSpec((_N_EMBED, _EMBED_DIM), lambda i: (0, 0)),
        ],
        out_specs=[
            pl.BlockSpec((1, 1, _BLOCK_M),
                         lambda i: (jnp.maximum(i - 1, 0), 0, 0)),
            pl.BlockSpec((_BLOCK_M, _EMBED_DIM),
                         lambda i: (jnp.maximum(i - 1, 0), 0)),
            pl.BlockSpec((_BLOCK_M, 2 * _EMBED_DIM),
                         lambda i: (jnp.maximum(i - 1, 0), 0)),
        ],
        out_shape=[
            jax.ShapeDtypeStruct((grid, 1, _BLOCK_M), jnp.int32),
            jax.ShapeDtypeStruct((n_rows, _EMBED_DIM), jnp.float32),
            jax.ShapeDtypeStruct((n_rows, 2 * _EMBED_DIM), jnp.float32),
        ],
        scratch_shapes=[
            pltpu.VMEM((_N_EMBED, _BLOCK_M), jnp.float32),
            pltpu.VMEM((1, _BLOCK_M), jnp.int32),
            pltpu.VMEM((_BLOCK_M, _EMBED_DIM), jnp.float32),
        ],
    )(flat, x2, embedding, emb2t)

    quantized = quant.reshape(inputs.shape)
    codes_out = codes.reshape(lead_shape + (2 * _EMBED_DIM,))
    encoding_indices = idx3.reshape(lead_shape)
    return (quantized, codes_out, encoding_indices)
